# baseline (speedup 1.0000x reference)
import functools
import numpy as np
import jax
import jax.numpy as jnp
from jax import lax
from jax.experimental import pallas as pl
from jax.experimental.pallas import tpu as pltpu
from jax.experimental.pallas import tpu_sc as plsc

_B = 4096
_NJ = 14
_COL = 14
_HW = _COL * _COL
_ROWS = _B * _NJ
_RB = 1792
_GRID = _ROWS // _RB
_SCALE = 1.0 / _COL
_OFLAT = _B * 2 * _NJ * _HW
_O2OFF = _NJ * _HW

_NW = 32
_PT = _ROWS // _NW
_NL = 16
_NK = _PT // _NL
_CHUNK = 128
_NCH = _PT // _CHUNK
_NMETA = 8
_NCHK = 16
_PCH = 2 * _NJ * _B // _NW // _NCHK
_RCH = _PT // _NCHK
_NCHK2 = 8
_PCH2 = 2 * _NJ * _B // _NW // _NCHK2
_OB = 16
_OLANE = 256


def _gauss_mat():
    rad = 4
    x = np.arange(-rad, rad + 1)
    phi = np.exp(-0.5 * x * x)
    phi = phi / phi.sum()
    eye = np.eye(_COL, dtype=np.float64)
    ap = np.pad(eye, ((0, 0), (rad, rad)), mode='symmetric')
    g = np.zeros((_COL, _COL))
    for i in range(2 * rad + 1):
        g = g + phi[i] * ap[:, i:i + _COL]
    return g


_G = _gauss_mat()
_AY = np.repeat(_G, _COL, axis=1).astype(np.float32)
_BX = np.tile(_G, (1, _COL)).astype(np.float32)


def _stage_a_body(h_ref, t_ref, v_ref, ay_ref, bx_ref,
                  meta_ref, acc_ref, sv_ref):
    pid = pl.program_id(0)
    h = h_ref[...].reshape(_RB, _HW)
    t = t_ref[...].reshape(_RB, 2)
    v = v_ref[...].reshape(_RB, 2)

    lane = lax.broadcasted_iota(jnp.int32, (_RB, _HW), 1)

    hmax = jnp.max(h, axis=1, keepdims=True)
    am = jnp.min(jnp.where(h == hmax, lane, _HW), axis=1, keepdims=True)
    yc = am // _COL
    xc = am - yc * _COL

    row = pid * _RB + lax.broadcasted_iota(jnp.int32, (_RB, 1), 0)
    b = row // _NJ
    j = row - b * _NJ
    base196 = 2 * _NJ * b + j

    g = jnp.where(hmax > 0.5, jnp.float32(_SCALE), jnp.float32(0.0))
    t0 = t[:, 0:1]
    t1 = t[:, 1:2]
    vv0 = v[:, 0:1] * v[:, 0:1]
    vv1 = v[:, 1:2] * v[:, 1:2]
    u0 = g * xc.astype(jnp.float32) - t0
    u1 = g * yc.astype(jnp.float32) - t1
    c1x = 2.0 * g * vv0 * u0
    c2x = vv0 * g * g
    c1y = 2.0 * g * vv1 * u1
    c2y = vv1 * g * g
    d2_const = jnp.sum(vv0 * u0 * u0 + vv1 * u1 * u1)

    zz = jnp.zeros((_RB, 1), jnp.float32)
    x_cols = jnp.concatenate(
        [am.astype(jnp.float32), base196.astype(jnp.float32),
         c1x, c2x, c1y, c2y, zz, zz], axis=1)
    meta_ref[...] = x_cols.T

    ti = t * jnp.float32(_COL)
    xi = jnp.clip(ti[:, 0:1].astype(jnp.int32), 0, _COL - 1)
    yi = jnp.clip(ti[:, 1:2].astype(jnp.int32), 0, _COL - 1)
    lane14 = lax.broadcasted_iota(jnp.int32, (_RB, _COL), 1)
    oh_y = (lane14 == yi).astype(jnp.float32)
    oh_x = (lane14 == xi).astype(jnp.float32)
    a_row = jnp.dot(oh_y, ay_ref[...], preferred_element_type=jnp.float32)
    b_row = jnp.dot(oh_x, bx_ref[...], preferred_element_type=jnp.float32)
    w = a_row * b_row
    wmn = jnp.min(w, axis=1, keepdims=True)
    wmx = jnp.max(w, axis=1, keepdims=True)
    den = wmx - wmn
    den = jnp.where(den == 0.0, jnp.float32(1.0), den)
    norm = (w - wmn) / den

    vis = v[:, 0:1] == 1.0
    diff = h - jnp.where(vis, norm, jnp.float32(0.0))
    keep = jnp.logical_or(vis, lane >= _COL)
    diff = jnp.where(keep, diff, jnp.float32(0.0))

    blk = (jnp.sum(diff * diff) + d2_const).reshape(1, 1)
    blk_sv = jnp.sum(v).reshape(1, 1)

    @pl.when(pid == 0)
    def _init():
        acc_ref[...] = jnp.zeros((1, 1), jnp.float32)
        sv_ref[...] = jnp.zeros((1, 1), jnp.float32)

    acc_ref[...] += blk
    sv_ref[...] += blk_sv


def _stage_a(h2, t2, v2):
    ay = jnp.asarray(_AY)
    bx = jnp.asarray(_BX)
    out_shapes = (
        jax.ShapeDtypeStruct((_GRID * _NMETA, _RB), jnp.float32),
        jax.ShapeDtypeStruct((1, 1), jnp.float32),
        jax.ShapeDtypeStruct((1, 1), jnp.float32),
    )
    return pl.pallas_call(
        _stage_a_body,
        grid=(_GRID,),
        in_specs=[
            pl.BlockSpec((_RB, _COL, _COL), lambda i: (i, 0, 0)),
            pl.BlockSpec((_RB // _NJ, _NJ, 2), lambda i: (i, 0, 0)),
            pl.BlockSpec((_RB // _NJ, _NJ, 2), lambda i: (i, 0, 0)),
            pl.BlockSpec((_COL, _HW), lambda i: (0, 0)),
            pl.BlockSpec((_COL, _HW), lambda i: (0, 0)),
        ],
        out_specs=(
            pl.BlockSpec((_NMETA, _RB), lambda i: (i, 0)),
            pl.BlockSpec((1, 1), lambda i: (0, 0)),
            pl.BlockSpec((1, 1), lambda i: (0, 0)),
        ),
        out_shape=out_shapes,
    )(h2, t2, v2, ay, bx)


def _odense_body(o_ref, out_ref):
    x = o_ref[...].reshape(_OB * 2 * _NJ, _HW)
    out_ref[...] = jnp.concatenate(
        [x, jnp.zeros((_OB * 2 * _NJ, _OLANE - _HW), jnp.float32)], axis=1)


def _odense(o4):
    return pl.pallas_call(
        _odense_body,
        grid=(_B // _OB,),
        in_specs=[pl.BlockSpec((_OB, 2 * _NJ, _COL, _COL),
                               lambda i: (i, 0, 0, 0))],
        out_specs=pl.BlockSpec((_OB * 2 * _NJ, _OLANE), lambda i: (i, 0)),
        out_shape=jax.ShapeDtypeStruct((2 * _ROWS, _OLANE), jnp.float32),
    )(o4)


def _sc_stage(o_flat, meta):
    mesh = plsc.VectorSubcoreMesh(core_axis_name="c", subcore_axis_name="s")

    @functools.partial(
        pl.kernel,
        mesh=mesh,
        out_type=jax.ShapeDtypeStruct((_NW, _NL), jnp.float32),
        scratch_types=[
            pltpu.VMEM((_NMETA, _PT), jnp.float32),
            pltpu.VMEM((_NCH, _CHUNK), jnp.int32),
            pltpu.VMEM((_NCH, _CHUNK), jnp.int32),
            pltpu.VMEM((_NCH, _CHUNK), jnp.float32),
            pltpu.VMEM((_NCH, _CHUNK), jnp.float32),
            pltpu.VMEM((_NL,), jnp.float32),
            pltpu.SemaphoreType.DMA,
        ],
    )
    def sc_k(o_hbm, meta_hbm, part_hbm,
             meta_v, idx1_v, idx2_v, val1_v, val2_v, acc_v, sem):
        wid = lax.axis_index("s") * 2 + lax.axis_index("c")
        pltpu.sync_copy(meta_hbm.at[pl.ds(wid * _NMETA, _NMETA)], meta_v)

        for k in range(_NK):
            sl = pl.ds(k * _NL, _NL)
            am = meta_v[0, sl].astype(jnp.int32)
            b196 = meta_v[1, sl].astype(jnp.int32)
            i1 = b196 * _OLANE + am
            r, c = divmod(k * _NL, _CHUNK)
            csl = pl.ds(c, _NL)
            idx1_v[r, csl] = i1
            idx2_v[r, csl] = i1 + _NJ * _OLANE

        copies = []
        for r in range(_NCH):
            copies.append(
                pltpu.async_copy(o_hbm.at[idx1_v.at[r]], val1_v.at[r], sem))
            copies.append(
                pltpu.async_copy(o_hbm.at[idx2_v.at[r]], val2_v.at[r], sem))
        for cp in copies:
            cp.wait()

        acc = jnp.zeros((_NL,), jnp.float32)
        for k in range(_NK):
            sl = pl.ds(k * _NL, _NL)
            r, c = divmod(k * _NL, _CHUNK)
            csl = pl.ds(c, _NL)
            ox = val1_v[r, csl]
            oy = val2_v[r, csl]
            acc = acc + ((meta_v[3, sl] * ox + meta_v[2, sl]) * ox
                         + (meta_v[5, sl] * oy + meta_v[4, sl]) * oy)
        acc_v[...] = acc
        pltpu.sync_copy(acc_v, part_hbm.at[wid])

    return sc_k(o_flat, meta)


def _stage_c_body(part_ref, acc_ref, sv_ref, out_ref):
    d2 = jnp.sum(part_ref[...]).reshape(1, 1)
    n1 = sv_ref[...] * jnp.float32(0.5)
    out_ref[...] = (acc_ref[...] + d2) / n1


def _stage_c(part, acc, sv):
    one = pl.BlockSpec((1, 1), lambda: (0, 0))
    return pl.pallas_call(
        _stage_c_body,
        grid=(),
        in_specs=[pl.BlockSpec((_NW, _NL), lambda: (0, 0)), one, one],
        out_specs=one,
        out_shape=jax.ShapeDtypeStruct((1, 1), jnp.float32),
    )(part, acc, sv)


def kernel(o, h, t, v):
    h3 = h.reshape(_ROWS, _COL, _COL)
    meta, acc, sv = _stage_a(h3, t, v)
    od = _odense(o)
    part = _sc_stage(od.reshape(2 * _ROWS * _OLANE), meta)
    out = _stage_c(part, acc, sv)
    return out[0, 0]


# speedup vs baseline: 1.5688x; 1.5688x over previous
import functools
import numpy as np
import jax
import jax.numpy as jnp
from jax import lax
from jax.experimental import pallas as pl
from jax.experimental.pallas import tpu as pltpu
from jax.experimental.pallas import tpu_sc as plsc

_B = 4096
_NJ = 14
_COL = 14
_HW = _COL * _COL
_ROWS = _B * _NJ
_RB = 1792
_GRID = _ROWS // _RB
_SCALE = 1.0 / _COL
_OFLAT = _B * 2 * _NJ * _HW
_O2OFF = _NJ * _HW

_NW = 32
_PT = _ROWS // _NW
_NL = 16
_NK = _PT // _NL
_CHUNK = 128
_NCH = _PT // _CHUNK
_NMETA = 8
_NCHK = 16
_PCH = 2 * _NJ * _B // _NW // _NCHK
_RCH = _PT // _NCHK
_NCHK2 = 8
_PCH2 = 2 * _NJ * _B // _NW // _NCHK2
_OB = 512
_OLANE = 256


def _gauss_mat():
    rad = 4
    x = np.arange(-rad, rad + 1)
    phi = np.exp(-0.5 * x * x)
    phi = phi / phi.sum()
    eye = np.eye(_COL, dtype=np.float64)
    ap = np.pad(eye, ((0, 0), (rad, rad)), mode='symmetric')
    g = np.zeros((_COL, _COL))
    for i in range(2 * rad + 1):
        g = g + phi[i] * ap[:, i:i + _COL]
    return g


_G = _gauss_mat()
_AY = np.repeat(_G, _COL, axis=1).astype(np.float32)
_BX = np.tile(_G, (1, _COL)).astype(np.float32)


def _stage_a_body(h_ref, t_ref, v_ref, ay_ref, bx_ref,
                  meta_ref, acc_ref, sv_ref):
    pid = pl.program_id(0)
    h = h_ref[...].reshape(_RB, _HW)
    t = t_ref[...]
    v = v_ref[...]

    lane = lax.broadcasted_iota(jnp.int32, (_RB, _HW), 1)

    hmax = jnp.max(h, axis=1, keepdims=True)
    am = jnp.min(jnp.where(h == hmax, lane, _HW), axis=1, keepdims=True)
    yc = am // _COL
    xc = am - yc * _COL

    row = pid * _RB + lax.broadcasted_iota(jnp.int32, (_RB, 1), 0)
    b = row // _NJ
    j = row - b * _NJ
    base196 = 2 * _NJ * b + j

    g = jnp.where(hmax > 0.5, jnp.float32(_SCALE), jnp.float32(0.0))
    t0 = t[:, 0:1]
    t1 = t[:, 1:2]
    vv0 = v[:, 0:1] * v[:, 0:1]
    vv1 = v[:, 1:2] * v[:, 1:2]
    u0 = g * xc.astype(jnp.float32) - t0
    u1 = g * yc.astype(jnp.float32) - t1
    c1x = 2.0 * g * vv0 * u0
    c2x = vv0 * g * g
    c1y = 2.0 * g * vv1 * u1
    c2y = vv1 * g * g
    d2_const = jnp.sum(vv0 * u0 * u0 + vv1 * u1 * u1)

    zz = jnp.zeros((_RB, 1), jnp.float32)
    x_cols = jnp.concatenate(
        [am.astype(jnp.float32), base196.astype(jnp.float32),
         c1x, c2x, c1y, c2y, zz, zz], axis=1)
    meta_ref[...] = x_cols.T

    ti = t * jnp.float32(_COL)
    xi = jnp.clip(ti[:, 0:1].astype(jnp.int32), 0, _COL - 1)
    yi = jnp.clip(ti[:, 1:2].astype(jnp.int32), 0, _COL - 1)
    lane14 = lax.broadcasted_iota(jnp.int32, (_RB, _COL), 1)
    oh_y = (lane14 == yi).astype(jnp.float32)
    oh_x = (lane14 == xi).astype(jnp.float32)
    a_row = jnp.dot(oh_y, ay_ref[...], preferred_element_type=jnp.float32)
    b_row = jnp.dot(oh_x, bx_ref[...], preferred_element_type=jnp.float32)
    w = a_row * b_row
    wmn = jnp.min(w, axis=1, keepdims=True)
    wmx = jnp.max(w, axis=1, keepdims=True)
    den = wmx - wmn
    den = jnp.where(den == 0.0, jnp.float32(1.0), den)
    norm = (w - wmn) / den

    vis = v[:, 0:1] == 1.0
    diff = h - jnp.where(vis, norm, jnp.float32(0.0))
    keep = jnp.logical_or(vis, lane >= _COL)
    diff = jnp.where(keep, diff, jnp.float32(0.0))

    blk = (jnp.sum(diff * diff) + d2_const).reshape(1, 1)
    blk_sv = jnp.sum(v).reshape(1, 1)

    @pl.when(pid == 0)
    def _init():
        acc_ref[...] = jnp.zeros((1, 1), jnp.float32)
        sv_ref[...] = jnp.zeros((1, 1), jnp.float32)

    acc_ref[...] += blk
    sv_ref[...] += blk_sv


def _stage_a(h2, t2, v2):
    ay = jnp.asarray(_AY)
    bx = jnp.asarray(_BX)
    out_shapes = (
        jax.ShapeDtypeStruct((_GRID * _NMETA, _RB), jnp.float32),
        jax.ShapeDtypeStruct((1, 1), jnp.float32),
        jax.ShapeDtypeStruct((1, 1), jnp.float32),
    )
    return pl.pallas_call(
        _stage_a_body,
        grid=(_GRID,),
        in_specs=[
            pl.BlockSpec((_RB, _COL, _COL), lambda i: (i, 0, 0)),
            pl.BlockSpec((_RB, 2), lambda i: (i, 0)),
            pl.BlockSpec((_RB, 2), lambda i: (i, 0)),
            pl.BlockSpec((_COL, _HW), lambda i: (0, 0)),
            pl.BlockSpec((_COL, _HW), lambda i: (0, 0)),
        ],
        out_specs=(
            pl.BlockSpec((_NMETA, _RB), lambda i: (i, 0)),
            pl.BlockSpec((1, 1), lambda i: (0, 0)),
            pl.BlockSpec((1, 1), lambda i: (0, 0)),
        ),
        out_shape=out_shapes,
    )(h2, t2, v2, ay, bx)


def _odense_body(o_ref, out_ref):
    x = o_ref[...].reshape(_OB, _HW)
    out_ref[...] = jnp.concatenate(
        [x, jnp.zeros((_OB, _OLANE - _HW), jnp.float32)], axis=1)


def _odense(o3):
    return pl.pallas_call(
        _odense_body,
        grid=(2 * _ROWS // _OB,),
        in_specs=[pl.BlockSpec((_OB, _COL, _COL), lambda i: (i, 0, 0))],
        out_specs=pl.BlockSpec((_OB, _OLANE), lambda i: (i, 0)),
        out_shape=jax.ShapeDtypeStruct((2 * _ROWS, _OLANE), jnp.float32),
    )(o3)


def _sc_stage(o_flat, meta):
    mesh = plsc.VectorSubcoreMesh(core_axis_name="c", subcore_axis_name="s")

    @functools.partial(
        pl.kernel,
        mesh=mesh,
        out_type=jax.ShapeDtypeStruct((_NW, _NL), jnp.float32),
        scratch_types=[
            pltpu.VMEM((_NMETA, _PT), jnp.float32),
            pltpu.VMEM((_NCH, _CHUNK), jnp.int32),
            pltpu.VMEM((_NCH, _CHUNK), jnp.int32),
            pltpu.VMEM((_NCH, _CHUNK), jnp.float32),
            pltpu.VMEM((_NCH, _CHUNK), jnp.float32),
            pltpu.VMEM((_NL,), jnp.float32),
            pltpu.SemaphoreType.DMA,
        ],
    )
    def sc_k(o_hbm, meta_hbm, part_hbm,
             meta_v, idx1_v, idx2_v, val1_v, val2_v, acc_v, sem):
        wid = lax.axis_index("s") * 2 + lax.axis_index("c")
        pltpu.sync_copy(meta_hbm.at[pl.ds(wid * _NMETA, _NMETA)], meta_v)

        for k in range(_NK):
            sl = pl.ds(k * _NL, _NL)
            am = meta_v[0, sl].astype(jnp.int32)
            b196 = meta_v[1, sl].astype(jnp.int32)
            i1 = b196 * _OLANE + am
            r, c = divmod(k * _NL, _CHUNK)
            csl = pl.ds(c, _NL)
            idx1_v[r, csl] = i1
            idx2_v[r, csl] = i1 + _NJ * _OLANE

        copies = []
        for r in range(_NCH):
            copies.append(
                pltpu.async_copy(o_hbm.at[idx1_v.at[r]], val1_v.at[r], sem))
            copies.append(
                pltpu.async_copy(o_hbm.at[idx2_v.at[r]], val2_v.at[r], sem))
        for cp in copies:
            cp.wait()

        acc = jnp.zeros((_NL,), jnp.float32)
        for k in range(_NK):
            sl = pl.ds(k * _NL, _NL)
            r, c = divmod(k * _NL, _CHUNK)
            csl = pl.ds(c, _NL)
            ox = val1_v[r, csl]
            oy = val2_v[r, csl]
            acc = acc + ((meta_v[3, sl] * ox + meta_v[2, sl]) * ox
                         + (meta_v[5, sl] * oy + meta_v[4, sl]) * oy)
        acc_v[...] = acc
        pltpu.sync_copy(acc_v, part_hbm.at[wid])

    return sc_k(o_flat, meta)


def _stage_c_body(part_ref, acc_ref, sv_ref, out_ref):
    d2 = jnp.sum(part_ref[...]).reshape(1, 1)
    n1 = sv_ref[...] * jnp.float32(0.5)
    out_ref[...] = (acc_ref[...] + d2) / n1


def _stage_c(part, acc, sv):
    one = pl.BlockSpec((1, 1), lambda: (0, 0))
    return pl.pallas_call(
        _stage_c_body,
        grid=(),
        in_specs=[pl.BlockSpec((_NW, _NL), lambda: (0, 0)), one, one],
        out_specs=one,
        out_shape=jax.ShapeDtypeStruct((1, 1), jnp.float32),
    )(part, acc, sv)


def kernel(o, h, t, v):
    h3 = h.reshape(_ROWS, _COL, _COL)
    t2 = t.reshape(_ROWS, 2)
    v2 = v.reshape(_ROWS, 2)
    meta, acc, sv = _stage_a(h3, t2, v2)
    od = _odense(o.reshape(2 * _ROWS, _COL, _COL))
    part = _sc_stage(od.reshape(2 * _ROWS * _OLANE), meta)
    out = _stage_c(part, acc, sv)
    return out[0, 0]


# speedup vs baseline: 1.7125x; 1.0916x over previous
import functools
import numpy as np
import jax
import jax.numpy as jnp
from jax import lax
from jax.experimental import pallas as pl
from jax.experimental.pallas import tpu as pltpu
from jax.experimental.pallas import tpu_sc as plsc

_B = 4096
_NJ = 14
_COL = 14
_HW = _COL * _COL
_ROWS = _B * _NJ
_RB = 1792
_GRID = _ROWS // _RB
_SCALE = 1.0 / _COL
_OFLAT = _B * 2 * _NJ * _HW
_O2OFF = _NJ * _HW

_NW = 32
_PT = _ROWS // _NW
_NL = 16
_NK = _PT // _NL
_CHUNK = 128
_NCH = _PT // _CHUNK
_NMETA = 8
_NCHK = 16
_PCH = 2 * _NJ * _B // _NW // _NCHK
_RCH = _PT // _NCHK
_NCHK2 = 8
_PCH2 = 2 * _NJ * _B // _NW // _NCHK2
_OB = 2048
_OLANE = 256


def _gauss_mat():
    rad = 4
    x = np.arange(-rad, rad + 1)
    phi = np.exp(-0.5 * x * x)
    phi = phi / phi.sum()
    eye = np.eye(_COL, dtype=np.float64)
    ap = np.pad(eye, ((0, 0), (rad, rad)), mode='symmetric')
    g = np.zeros((_COL, _COL))
    for i in range(2 * rad + 1):
        g = g + phi[i] * ap[:, i:i + _COL]
    return g


_G = _gauss_mat()
_AY = np.repeat(_G, _COL, axis=1).astype(np.float32)
_BX = np.tile(_G, (1, _COL)).astype(np.float32)


def _stage_a_body(h_ref, t_ref, v_ref, ay_ref, bx_ref,
                  meta_ref, acc_ref, sv_ref):
    pid = pl.program_id(0)
    h = h_ref[...].reshape(_RB, _HW)
    t = t_ref[...]
    v = v_ref[...]

    lane = lax.broadcasted_iota(jnp.int32, (_RB, _HW), 1)

    hmax = jnp.max(h, axis=1, keepdims=True)
    am = jnp.min(jnp.where(h == hmax, lane, _HW), axis=1, keepdims=True)
    yc = am // _COL
    xc = am - yc * _COL

    row = pid * _RB + lax.broadcasted_iota(jnp.int32, (_RB, 1), 0)
    b = row // _NJ
    j = row - b * _NJ
    base196 = 2 * _NJ * b + j

    g = jnp.where(hmax > 0.5, jnp.float32(_SCALE), jnp.float32(0.0))
    t0 = t[:, 0:1]
    t1 = t[:, 1:2]
    vv0 = v[:, 0:1] * v[:, 0:1]
    vv1 = v[:, 1:2] * v[:, 1:2]
    u0 = g * xc.astype(jnp.float32) - t0
    u1 = g * yc.astype(jnp.float32) - t1
    c1x = 2.0 * g * vv0 * u0
    c2x = vv0 * g * g
    c1y = 2.0 * g * vv1 * u1
    c2y = vv1 * g * g
    d2_const = jnp.sum(vv0 * u0 * u0 + vv1 * u1 * u1)

    zz = jnp.zeros((_RB, 1), jnp.float32)
    x_cols = jnp.concatenate(
        [am.astype(jnp.float32), base196.astype(jnp.float32),
         c1x, c2x, c1y, c2y, zz, zz], axis=1)
    meta_ref[...] = x_cols.T

    ti = t * jnp.float32(_COL)
    xi = jnp.clip(ti[:, 0:1].astype(jnp.int32), 0, _COL - 1)
    yi = jnp.clip(ti[:, 1:2].astype(jnp.int32), 0, _COL - 1)
    lane14 = lax.broadcasted_iota(jnp.int32, (_RB, _COL), 1)
    oh_y = (lane14 == yi).astype(jnp.float32)
    oh_x = (lane14 == xi).astype(jnp.float32)
    a_row = jnp.dot(oh_y, ay_ref[...], preferred_element_type=jnp.float32)
    b_row = jnp.dot(oh_x, bx_ref[...], preferred_element_type=jnp.float32)
    w = a_row * b_row
    wmn = jnp.min(w, axis=1, keepdims=True)
    wmx = jnp.max(w, axis=1, keepdims=True)
    den = wmx - wmn
    den = jnp.where(den == 0.0, jnp.float32(1.0), den)
    norm = (w - wmn) / den

    vis = v[:, 0:1] == 1.0
    diff = h - jnp.where(vis, norm, jnp.float32(0.0))
    keep = jnp.logical_or(vis, lane >= _COL)
    diff = jnp.where(keep, diff, jnp.float32(0.0))

    blk = (jnp.sum(diff * diff) + d2_const).reshape(1, 1)
    blk_sv = jnp.sum(v).reshape(1, 1)

    @pl.when(pid == 0)
    def _init():
        acc_ref[...] = jnp.zeros((1, 1), jnp.float32)
        sv_ref[...] = jnp.zeros((1, 1), jnp.float32)

    acc_ref[...] += blk
    sv_ref[...] += blk_sv


def _stage_a(h2, t2, v2):
    ay = jnp.asarray(_AY)
    bx = jnp.asarray(_BX)
    out_shapes = (
        jax.ShapeDtypeStruct((_GRID * _NMETA, _RB), jnp.float32),
        jax.ShapeDtypeStruct((1, 1), jnp.float32),
        jax.ShapeDtypeStruct((1, 1), jnp.float32),
    )
    return pl.pallas_call(
        _stage_a_body,
        grid=(_GRID,),
        in_specs=[
            pl.BlockSpec((_RB, _COL, _COL), lambda i: (i, 0, 0)),
            pl.BlockSpec((_RB, 2), lambda i: (i, 0)),
            pl.BlockSpec((_RB, 2), lambda i: (i, 0)),
            pl.BlockSpec((_COL, _HW), lambda i: (0, 0)),
            pl.BlockSpec((_COL, _HW), lambda i: (0, 0)),
        ],
        out_specs=(
            pl.BlockSpec((_NMETA, _RB), lambda i: (i, 0)),
            pl.BlockSpec((1, 1), lambda i: (0, 0)),
            pl.BlockSpec((1, 1), lambda i: (0, 0)),
        ),
        out_shape=out_shapes,
    )(h2, t2, v2, ay, bx)


def _odense_body(o_ref, out_ref):
    x = o_ref[...].reshape(_OB, _HW)
    out_ref[...] = jnp.concatenate(
        [x, jnp.zeros((_OB, _OLANE - _HW), jnp.float32)], axis=1)


def _odense(o3):
    return pl.pallas_call(
        _odense_body,
        grid=(2 * _ROWS // _OB,),
        in_specs=[pl.BlockSpec((_OB, _COL, _COL), lambda i: (i, 0, 0))],
        out_specs=pl.BlockSpec((_OB, _OLANE), lambda i: (i, 0)),
        out_shape=jax.ShapeDtypeStruct((2 * _ROWS, _OLANE), jnp.float32),
    )(o3)


def _sc_stage(o_flat, meta):
    mesh = plsc.VectorSubcoreMesh(core_axis_name="c", subcore_axis_name="s")

    @functools.partial(
        pl.kernel,
        mesh=mesh,
        out_type=jax.ShapeDtypeStruct((_NW, _NL), jnp.float32),
        scratch_types=[
            pltpu.VMEM((_NMETA, _PT), jnp.float32),
            pltpu.VMEM((_NCH, _CHUNK), jnp.int32),
            pltpu.VMEM((_NCH, _CHUNK), jnp.int32),
            pltpu.VMEM((_NCH, _CHUNK), jnp.float32),
            pltpu.VMEM((_NCH, _CHUNK), jnp.float32),
            pltpu.VMEM((_NL,), jnp.float32),
            pltpu.SemaphoreType.DMA,
        ],
    )
    def sc_k(o_hbm, meta_hbm, part_hbm,
             meta_v, idx1_v, idx2_v, val1_v, val2_v, acc_v, sem):
        wid = lax.axis_index("s") * 2 + lax.axis_index("c")
        pltpu.sync_copy(meta_hbm.at[pl.ds(wid * _NMETA, _NMETA)], meta_v)

        for k in range(_NK):
            sl = pl.ds(k * _NL, _NL)
            am = meta_v[0, sl].astype(jnp.int32)
            b196 = meta_v[1, sl].astype(jnp.int32)
            i1 = b196 * _OLANE + am
            r, c = divmod(k * _NL, _CHUNK)
            csl = pl.ds(c, _NL)
            idx1_v[r, csl] = i1
            idx2_v[r, csl] = i1 + _NJ * _OLANE

        copies = []
        for r in range(_NCH):
            copies.append(
                pltpu.async_copy(o_hbm.at[idx1_v.at[r]], val1_v.at[r], sem))
            copies.append(
                pltpu.async_copy(o_hbm.at[idx2_v.at[r]], val2_v.at[r], sem))
        for cp in copies:
            cp.wait()

        acc = jnp.zeros((_NL,), jnp.float32)
        for k in range(_NK):
            sl = pl.ds(k * _NL, _NL)
            r, c = divmod(k * _NL, _CHUNK)
            csl = pl.ds(c, _NL)
            ox = val1_v[r, csl]
            oy = val2_v[r, csl]
            acc = acc + ((meta_v[3, sl] * ox + meta_v[2, sl]) * ox
                         + (meta_v[5, sl] * oy + meta_v[4, sl]) * oy)
        acc_v[...] = acc
        pltpu.sync_copy(acc_v, part_hbm.at[wid])

    return sc_k(o_flat, meta)


def _stage_c_body(part_ref, acc_ref, sv_ref, out_ref):
    d2 = jnp.sum(part_ref[...]).reshape(1, 1)
    n1 = sv_ref[...] * jnp.float32(0.5)
    out_ref[...] = (acc_ref[...] + d2) / n1


def _stage_c(part, acc, sv):
    one = pl.BlockSpec((1, 1), lambda: (0, 0))
    return pl.pallas_call(
        _stage_c_body,
        grid=(),
        in_specs=[pl.BlockSpec((_NW, _NL), lambda: (0, 0)), one, one],
        out_specs=one,
        out_shape=jax.ShapeDtypeStruct((1, 1), jnp.float32),
    )(part, acc, sv)


def kernel(o, h, t, v):
    h3 = h.reshape(_ROWS, _COL, _COL)
    t2 = t.reshape(_ROWS, 2)
    v2 = v.reshape(_ROWS, 2)
    meta, acc, sv = _stage_a(h3, t2, v2)
    od = _odense(o.reshape(2 * _ROWS, _COL, _COL))
    part = _sc_stage(od.reshape(2 * _ROWS * _OLANE), meta)
    out = _stage_c(part, acc, sv)
    return out[0, 0]
